# Initial kernel scaffold; baseline (speedup 1.0000x reference)
import jax
import jax.numpy as jnp
from jax.experimental import pallas as pl


def kernel(nodes, to_neighs, features, weight):
    raise NotImplementedError("write your pallas kernel here")


# speedup vs baseline: 1.8461x; 1.8461x over previous
import functools

import jax
import jax.numpy as jnp
from jax import lax
from jax.experimental import pallas as pl
from jax.experimental.pallas import tpu as pltpu
from jax.experimental.pallas import tpu_sc as plsc

NC = 2
NS = 16
NW = NC * NS
LANES = 16


def _sc_gather_mean(features, neigh_idx, node_idx, *, B, K, D, C):
    rows_per_w = B // NW
    nchunk = rows_per_w // C
    mesh = plsc.VectorSubcoreMesh(
        core_axis_name="c", subcore_axis_name="s", num_cores=NC, num_subcores=NS
    )

    @functools.partial(
        pl.kernel,
        out_type=(
            jax.ShapeDtypeStruct((B, D), jnp.float32),
            jax.ShapeDtypeStruct((B, D), jnp.float32),
        ),
        mesh=mesh,
        scratch_types=[
            pltpu.VMEM((nchunk, C * K), jnp.int32),
            pltpu.VMEM((nchunk, C), jnp.int32),
            pltpu.VMEM((C * K, D), jnp.float32),
            pltpu.VMEM((C, D), jnp.float32),
            pltpu.VMEM((C, D), jnp.float32),
            pltpu.SemaphoreType.DMA,
            pltpu.SemaphoreType.DMA,
        ],
    )
    def k(feat_hbm, nidx_hbm, sidx_hbm, self_hbm, agg_hbm,
          nidx_v, sidx_v, gbuf, sbuf, obuf, gsem, ssem):
        wid = lax.axis_index("s") * NC + lax.axis_index("c")
        pltpu.sync_copy(nidx_hbm.at[wid], nidx_v)
        pltpu.sync_copy(sidx_hbm.at[wid], sidx_v)

        def chunk_body(chunk, carry):
            row_base = (wid * nchunk + chunk) * C
            ng = pltpu.async_copy(feat_hbm.at[nidx_v.at[chunk]], gbuf, gsem)
            sg = pltpu.async_copy(feat_hbm.at[sidx_v.at[chunk]], sbuf, ssem)
            ng.wait()
            sg.wait()
            for r in range(C):
                for d in range(D // LANES):
                    sl = pl.ds(d * LANES, LANES)
                    acc = gbuf[r * K, sl]
                    for kk in range(1, K):
                        acc = acc + gbuf[r * K + kk, sl]
                    obuf[r, sl] = acc * (1.0 / K)
            pltpu.sync_copy(sbuf, self_hbm.at[pl.ds(row_base, C)])
            pltpu.sync_copy(obuf, agg_hbm.at[pl.ds(row_base, C)])
            return carry

        lax.fori_loop(0, nchunk, chunk_body, 0)

    return k(features, neigh_idx, node_idx)


def _tc_matmul_relu(self_feats, agg_feats, w_top, w_bot):
    B, D = self_feats.shape
    E = w_top.shape[1]
    BLK = 512

    def mm(s_ref, a_ref, wt_ref, wb_ref, o_ref):
        acc = jnp.dot(s_ref[...], wt_ref[...], preferred_element_type=jnp.float32)
        acc = acc + jnp.dot(a_ref[...], wb_ref[...], preferred_element_type=jnp.float32)
        o_ref[...] = jnp.maximum(acc, 0.0)

    return pl.pallas_call(
        mm,
        grid=(B // BLK,),
        in_specs=[
            pl.BlockSpec((BLK, D), lambda i: (i, 0)),
            pl.BlockSpec((BLK, D), lambda i: (i, 0)),
            pl.BlockSpec((D, E), lambda i: (0, 0)),
            pl.BlockSpec((D, E), lambda i: (0, 0)),
        ],
        out_specs=pl.BlockSpec((BLK, E), lambda i: (i, 0)),
        out_shape=jax.ShapeDtypeStruct((B, E), jnp.float32),
    )(self_feats, agg_feats, w_top, w_bot)


def kernel(nodes, to_neighs, features, weight):
    B, K = to_neighs.shape
    D = features.shape[1]
    C = 4
    rows_per_w = B // NW
    nchunk = rows_per_w // C

    neigh_idx = to_neighs.astype(jnp.int32).reshape(NW, nchunk, C * K)
    node_idx = nodes.astype(jnp.int32).reshape(NW, nchunk, C)

    self_feats, agg_feats = _sc_gather_mean(
        features, neigh_idx, node_idx, B=B, K=K, D=D, C=C
    )
    w_top = weight[:D]
    w_bot = weight[D:]
    return _tc_matmul_relu(self_feats, agg_feats, w_top, w_bot)


# speedup vs baseline: 2.5314x; 1.3712x over previous
import functools

import jax
import jax.numpy as jnp
from jax import lax
from jax.experimental import pallas as pl
from jax.experimental.pallas import tpu as pltpu
from jax.experimental.pallas import tpu_sc as plsc

NC = 2
NS = 16
NW = NC * NS
LANES = 16


def _sc_gather_mean(features, neigh_idx, node_idx, *, B, K, D, C):
    rows_per_w = B // NW
    nchunk = rows_per_w // C
    assert nchunk % 2 == 0
    mesh = plsc.VectorSubcoreMesh(
        core_axis_name="c", subcore_axis_name="s", num_cores=NC, num_subcores=NS
    )

    @functools.partial(
        pl.kernel,
        out_type=(
            jax.ShapeDtypeStruct((B, D), jnp.float32),
            jax.ShapeDtypeStruct((B, D), jnp.float32),
        ),
        mesh=mesh,
        scratch_types=[
            pltpu.VMEM((nchunk, C * K), jnp.int32),
            pltpu.VMEM((nchunk, C), jnp.int32),
            pltpu.VMEM((2, C * K, D), jnp.float32),
            pltpu.VMEM((2, C, D), jnp.float32),
            pltpu.VMEM((2, C, D), jnp.float32),
            pltpu.SemaphoreType.DMA,
            pltpu.SemaphoreType.DMA,
            pltpu.SemaphoreType.DMA,
            pltpu.SemaphoreType.DMA,
            pltpu.SemaphoreType.DMA,
            pltpu.SemaphoreType.DMA,
            pltpu.SemaphoreType.DMA,
            pltpu.SemaphoreType.DMA,
        ],
    )
    def k(feat_hbm, nidx_hbm, sidx_hbm, self_hbm, agg_hbm,
          nidx_v, sidx_v, gbuf, sbuf, obuf,
          gsem0, gsem1, ssem0, ssem1, osem0, osem1, psem0, psem1):
        wid = lax.axis_index("s") * NC + lax.axis_index("c")
        gsems = (gsem0, gsem1)
        ssems = (ssem0, ssem1)
        osems = (osem0, osem1)
        psems = (psem0, psem1)
        pltpu.sync_copy(nidx_hbm.at[wid], nidx_v)
        pltpu.sync_copy(sidx_hbm.at[wid], sidx_v)

        def issue(i, slot):
            pltpu.async_copy(feat_hbm.at[nidx_v.at[i]], gbuf.at[slot], gsems[slot])
            pltpu.async_copy(feat_hbm.at[sidx_v.at[i]], sbuf.at[slot], ssems[slot])

        def g_wait(i, slot):
            pltpu.make_async_copy(feat_hbm.at[nidx_v.at[i]], gbuf.at[slot],
                                  gsems[slot]).wait()
            pltpu.make_async_copy(feat_hbm.at[sidx_v.at[i]], sbuf.at[slot],
                                  ssems[slot]).wait()

        def agg_out_wait(i, slot):
            row_base = (wid * nchunk + i) * C
            pltpu.make_async_copy(obuf.at[slot], agg_hbm.at[pl.ds(row_base, C)],
                                  osems[slot]).wait()

        def self_out_wait(i, slot):
            row_base = (wid * nchunk + i) * C
            pltpu.make_async_copy(sbuf.at[slot], self_hbm.at[pl.ds(row_base, C)],
                                  psems[slot]).wait()

        issue(0, 0)

        def pair_body(p, carry):
            for b in range(2):
                i = 2 * p + b
                nxt = i + 1

                @pl.when(nxt < nchunk)
                def _():
                    @pl.when(i >= 1)
                    def _():
                        self_out_wait(i - 1, 1 - b)

                    issue(nxt, 1 - b)

                g_wait(i, b)

                @pl.when(i >= 2)
                def _():
                    agg_out_wait(i - 2, b)

                for r in range(C):
                    for d in range(D // LANES):
                        sl = pl.ds(d * LANES, LANES)
                        acc = gbuf[b, r * K, sl]
                        for kk in range(1, K):
                            acc = acc + gbuf[b, r * K + kk, sl]
                        obuf[b, r, sl] = acc * (1.0 / K)

                row_base = (wid * nchunk + i) * C
                pltpu.async_copy(obuf.at[b], agg_hbm.at[pl.ds(row_base, C)],
                                 osems[b])
                pltpu.async_copy(sbuf.at[b], self_hbm.at[pl.ds(row_base, C)],
                                 psems[b])
            return carry

        lax.fori_loop(0, nchunk // 2, pair_body, 0)
        agg_out_wait(nchunk - 2, 0)
        agg_out_wait(nchunk - 1, 1)
        self_out_wait(nchunk - 2, 0)
        self_out_wait(nchunk - 1, 1)

    return k(features, neigh_idx, node_idx)


def _tc_matmul_relu(self_feats, agg_feats, w_top, w_bot):
    B, D = self_feats.shape
    E = w_top.shape[1]
    BLK = 512

    def mm(s_ref, a_ref, wt_ref, wb_ref, o_ref):
        acc = jnp.dot(s_ref[...], wt_ref[...], preferred_element_type=jnp.float32)
        acc = acc + jnp.dot(a_ref[...], wb_ref[...], preferred_element_type=jnp.float32)
        o_ref[...] = jnp.maximum(acc, 0.0)

    return pl.pallas_call(
        mm,
        grid=(B // BLK,),
        in_specs=[
            pl.BlockSpec((BLK, D), lambda i: (i, 0)),
            pl.BlockSpec((BLK, D), lambda i: (i, 0)),
            pl.BlockSpec((D, E), lambda i: (0, 0)),
            pl.BlockSpec((D, E), lambda i: (0, 0)),
        ],
        out_specs=pl.BlockSpec((BLK, E), lambda i: (i, 0)),
        out_shape=jax.ShapeDtypeStruct((B, E), jnp.float32),
    )(self_feats, agg_feats, w_top, w_bot)


def kernel(nodes, to_neighs, features, weight):
    B, K = to_neighs.shape
    D = features.shape[1]
    C = 4
    rows_per_w = B // NW
    nchunk = rows_per_w // C

    neigh_idx = to_neighs.astype(jnp.int32).reshape(NW, nchunk, C * K)
    node_idx = nodes.astype(jnp.int32).reshape(NW, nchunk, C)

    self_feats, agg_feats = _sc_gather_mean(
        features, neigh_idx, node_idx, B=B, K=K, D=D, C=C
    )
    w_top = weight[:D]
    w_bot = weight[D:]
    return _tc_matmul_relu(self_feats, agg_feats, w_top, w_bot)
